# baseline (speedup 1.0000x reference)
import jax
import jax.numpy as jnp
from jax import lax
from jax.experimental import pallas as pl
from jax.experimental.pallas import tpu as pltpu
from jax.experimental.pallas import tpu_sc as plsc

B = 4096
S = 200
D = 128
N = B * S
NC = 2
NS = 16
NW = NC * NS
ROWS_W = N // NW
CH = 128
NCH = ROWS_W // CH
NBUF = 4
BB = 64
EPS = 1e-12


def _gather_body(x_hbm, tab_hbm, out_hbm,
                 idx0, idx1, idx2, idx3, rows0, rows1, rows2, rows3,
                 gs0, gs1, gs2, gs3, ws0, ws1, ws2, ws3):
    idx_v = [idx0, idx1, idx2, idx3]
    rows_v = [rows0, rows1, rows2, rows3]
    gsem = [gs0, gs1, gs2, gs3]
    wsem = [ws0, ws1, ws2, ws3]

    wid = lax.axis_index("s") * NC + lax.axis_index("c")
    base_w = wid * ROWS_W

    def start_gather(cc, b):
        pltpu.sync_copy(x_hbm.at[pl.ds(base_w + cc * CH, CH)], idx_v[b])
        pltpu.async_copy(tab_hbm.at[idx_v[b]], rows_v[b], gsem[b])

    for b in range(NBUF - 1):
        start_gather(b, b)

    def ring(i, carry):
        for b in range(NBUF):
            cc = i * NBUF + b
            pb = (b - 1) % NBUF
            pltpu.make_async_copy(rows_v[b], out_hbm.at[pl.ds(0, CH)],
                                  gsem[b]).wait()
            pltpu.async_copy(rows_v[b],
                             out_hbm.at[pl.ds(base_w + cc * CH, CH)], wsem[b])

            @pl.when(cc >= 1)
            def _wait_prev_write():
                pltpu.make_async_copy(
                    rows_v[pb], out_hbm.at[pl.ds(0, CH)], wsem[pb]).wait()

            @pl.when(cc + NBUF - 1 < NCH)
            def _prefetch():
                start_gather(cc + NBUF - 1, pb)
        return carry

    lax.fori_loop(0, NCH // NBUF, ring, 0)
    pltpu.make_async_copy(rows_v[(NCH - 1) % NBUF], out_hbm.at[pl.ds(0, CH)],
                          wsem[(NCH - 1) % NBUF]).wait()


def _sc_gather(xf, id_table):
    mesh = plsc.VectorSubcoreMesh(core_axis_name="c", subcore_axis_name="s")
    run = pl.kernel(
        _gather_body,
        mesh=mesh,
        compiler_params=pltpu.CompilerParams(use_tc_tiling_on_sc=False),
        out_type=jax.ShapeDtypeStruct((N, D // 2), jnp.int32),
        scratch_types=(
            [pltpu.VMEM((CH,), jnp.int32) for _ in range(NBUF)]
            + [pltpu.VMEM((CH, D // 2), jnp.int32) for _ in range(NBUF)]
            + [pltpu.SemaphoreType.DMA for _ in range(2 * NBUF)]
        ),
    )
    return run(xf, id_table)


def _ln_body(t_ref, pos_ref, g_ref, b_ref, o_ref):
    ti = t_ref[...]
    t_lo = lax.bitcast_convert_type(
        lax.shift_left(ti, 16), jnp.float32)
    t_hi = lax.bitcast_convert_type(
        lax.bitwise_and(ti, jnp.int32(-65536)), jnp.float32)
    t = jnp.concatenate([t_lo, t_hi], axis=-1) + pos_ref[...]
    mean = jnp.mean(t, axis=-1, keepdims=True)
    var = jnp.mean(t * t, axis=-1, keepdims=True) - mean * mean
    y = lax.rsqrt(var + EPS)
    o_ref[...] = (t - mean) * y * g_ref[...] + b_ref[...]


def _tc_ln(t, pos_table, ln_gamma, ln_beta):
    pos3 = pos_table.reshape(1, S, D)
    g3 = ln_gamma.reshape(1, 1, D)
    b3 = ln_beta.reshape(1, 1, D)
    return pl.pallas_call(
        _ln_body,
        grid=(B // BB,),
        in_specs=[
            pl.BlockSpec((BB, S, D // 2), lambda i: (i, 0, 0)),
            pl.BlockSpec((1, S, D), lambda i: (0, 0, 0)),
            pl.BlockSpec((1, 1, D), lambda i: (0, 0, 0)),
            pl.BlockSpec((1, 1, D), lambda i: (0, 0, 0)),
        ],
        out_specs=pl.BlockSpec((BB, S, D), lambda i: (i, 0, 0)),
        out_shape=jax.ShapeDtypeStruct((B, S, D), jnp.float32),
    )(t, pos3, g3, b3)


def kernel(x, id_table, pos_table, ln_gamma, ln_beta):
    tab16 = id_table.astype(jnp.bfloat16)
    tab_pk = lax.bitcast_convert_type(
        jnp.stack([tab16[:, :D // 2], tab16[:, D // 2:]], axis=-1), jnp.int32)
    rows = _sc_gather(x.reshape(N), tab_pk)
    return _tc_ln(rows.reshape(B, S, D // 2), pos_table, ln_gamma, ln_beta)


# speedup vs baseline: 1.4551x; 1.4551x over previous
import jax
import jax.numpy as jnp
from jax import lax
from jax.experimental import pallas as pl
from jax.experimental.pallas import tpu as pltpu
from jax.experimental.pallas import tpu_sc as plsc

B = 4096
S = 200
D = 128
N = B * S
NC = 2
NS = 16
NW = NC * NS
ROWS_W = N // NW
CH = 128
NCH = ROWS_W // CH
NBUF = 4
BB = 128
SB = 40
EPS = 1e-12


def _gather_body(x_hbm, tab_hbm, out_hbm,
                 idx0, idx1, idx2, idx3, rows0, rows1, rows2, rows3,
                 gs0, gs1, gs2, gs3, ws0, ws1, ws2, ws3):
    idx_v = [idx0, idx1, idx2, idx3]
    rows_v = [rows0, rows1, rows2, rows3]
    gsem = [gs0, gs1, gs2, gs3]
    wsem = [ws0, ws1, ws2, ws3]

    wid = lax.axis_index("s") * NC + lax.axis_index("c")
    base_w = wid * ROWS_W

    def start_gather(cc, b):
        pltpu.sync_copy(x_hbm.at[pl.ds(base_w + cc * CH, CH)], idx_v[b])
        pltpu.async_copy(tab_hbm.at[idx_v[b]], rows_v[b], gsem[b])

    for b in range(NBUF - 1):
        start_gather(b, b)

    def ring(i, carry):
        for b in range(NBUF):
            cc = i * NBUF + b
            pb = (b - 1) % NBUF
            pltpu.make_async_copy(rows_v[b], out_hbm.at[pl.ds(0, CH)],
                                  gsem[b]).wait()
            pltpu.async_copy(rows_v[b],
                             out_hbm.at[pl.ds(base_w + cc * CH, CH)], wsem[b])

            @pl.when(cc >= 1)
            def _wait_prev_write():
                pltpu.make_async_copy(
                    rows_v[pb], out_hbm.at[pl.ds(0, CH)], wsem[pb]).wait()

            @pl.when(cc + NBUF - 1 < NCH)
            def _prefetch():
                start_gather(cc + NBUF - 1, pb)
        return carry

    lax.fori_loop(0, NCH // NBUF, ring, 0)
    pltpu.make_async_copy(rows_v[(NCH - 1) % NBUF], out_hbm.at[pl.ds(0, CH)],
                          wsem[(NCH - 1) % NBUF]).wait()


def _sc_gather(xf, id_table):
    mesh = plsc.VectorSubcoreMesh(core_axis_name="c", subcore_axis_name="s")
    run = pl.kernel(
        _gather_body,
        mesh=mesh,
        compiler_params=pltpu.CompilerParams(use_tc_tiling_on_sc=False),
        out_type=jax.ShapeDtypeStruct((N, D), jnp.float32),
        scratch_types=(
            [pltpu.VMEM((CH,), jnp.int32) for _ in range(NBUF)]
            + [pltpu.VMEM((CH, D), jnp.float32) for _ in range(NBUF)]
            + [pltpu.SemaphoreType.DMA for _ in range(2 * NBUF)]
        ),
    )
    return run(xf, id_table)


def _ln_body(t_ref, pos_ref, g_ref, b_ref, o_ref):
    t = t_ref[...] + pos_ref[...]
    mean = jnp.mean(t, axis=-1, keepdims=True)
    var = jnp.mean(t * t, axis=-1, keepdims=True) - mean * mean
    y = lax.rsqrt(var + EPS)
    o_ref[...] = (t - mean) * y * g_ref[...] + b_ref[...]


def _tc_ln(t, pos_table, ln_gamma, ln_beta):
    pos3 = pos_table.reshape(1, S, D)
    g3 = ln_gamma.reshape(1, 1, D)
    b3 = ln_beta.reshape(1, 1, D)
    return pl.pallas_call(
        _ln_body,
        grid=(B // BB, S // SB),
        in_specs=[
            pl.BlockSpec((BB, SB, D), lambda i, j: (i, j, 0)),
            pl.BlockSpec((1, SB, D), lambda i, j: (0, j, 0)),
            pl.BlockSpec((1, 1, D), lambda i, j: (0, 0, 0)),
            pl.BlockSpec((1, 1, D), lambda i, j: (0, 0, 0)),
        ],
        out_specs=pl.BlockSpec((BB, SB, D), lambda i, j: (i, j, 0)),
        out_shape=jax.ShapeDtypeStruct((B, S, D), jnp.float32),
    )(t, pos3, g3, b3)


def kernel(x, id_table, pos_table, ln_gamma, ln_beta):
    rows = _sc_gather(x.reshape(N), id_table)
    return _tc_ln(rows.reshape(B, S, D), pos_table, ln_gamma, ln_beta)


# speedup vs baseline: 1.5983x; 1.0984x over previous
import jax
import jax.numpy as jnp
from jax import lax
from jax.experimental import pallas as pl
from jax.experimental.pallas import tpu as pltpu
from jax.experimental.pallas import tpu_sc as plsc

B = 4096
S = 200
D = 128
N = B * S
NC = 2
NS = 16
NW = NC * NS
ROWS_W = N // NW
CH = 128
NCH = ROWS_W // CH
NBUF = 4
BB = 64
SB = 200
EPS = 1e-12


def _gather_body(x_hbm, tab_hbm, out_hbm,
                 idx0, idx1, idx2, idx3, rows0, rows1, rows2, rows3,
                 gs0, gs1, gs2, gs3, ws0, ws1, ws2, ws3):
    idx_v = [idx0, idx1, idx2, idx3]
    rows_v = [rows0, rows1, rows2, rows3]
    gsem = [gs0, gs1, gs2, gs3]
    wsem = [ws0, ws1, ws2, ws3]

    wid = lax.axis_index("s") * NC + lax.axis_index("c")
    base_w = wid * ROWS_W

    def start_gather(cc, b):
        pltpu.sync_copy(x_hbm.at[pl.ds(base_w + cc * CH, CH)], idx_v[b])
        pltpu.async_copy(tab_hbm.at[idx_v[b]], rows_v[b], gsem[b])

    for b in range(NBUF - 1):
        start_gather(b, b)

    def ring(i, carry):
        for b in range(NBUF):
            cc = i * NBUF + b
            pb = (b - 1) % NBUF
            pltpu.make_async_copy(rows_v[b], out_hbm.at[pl.ds(0, CH)],
                                  gsem[b]).wait()
            pltpu.async_copy(rows_v[b],
                             out_hbm.at[pl.ds(base_w + cc * CH, CH)], wsem[b])

            @pl.when(cc >= 1)
            def _wait_prev_write():
                pltpu.make_async_copy(
                    rows_v[pb], out_hbm.at[pl.ds(0, CH)], wsem[pb]).wait()

            @pl.when(cc + NBUF - 1 < NCH)
            def _prefetch():
                start_gather(cc + NBUF - 1, pb)
        return carry

    lax.fori_loop(0, NCH // NBUF, ring, 0)
    pltpu.make_async_copy(rows_v[(NCH - 1) % NBUF], out_hbm.at[pl.ds(0, CH)],
                          wsem[(NCH - 1) % NBUF]).wait()


def _sc_gather(xf, id_table):
    mesh = plsc.VectorSubcoreMesh(core_axis_name="c", subcore_axis_name="s")
    run = pl.kernel(
        _gather_body,
        mesh=mesh,
        compiler_params=pltpu.CompilerParams(use_tc_tiling_on_sc=False),
        out_type=jax.ShapeDtypeStruct((N, D), jnp.float32),
        scratch_types=(
            [pltpu.VMEM((CH,), jnp.int32) for _ in range(NBUF)]
            + [pltpu.VMEM((CH, D), jnp.float32) for _ in range(NBUF)]
            + [pltpu.SemaphoreType.DMA for _ in range(2 * NBUF)]
        ),
    )
    return run(xf, id_table)


def _ln_body(t_ref, pos_ref, g_ref, b_ref, o_ref):
    t = t_ref[...] + pos_ref[...]
    mean = jnp.mean(t, axis=-1, keepdims=True)
    var = jnp.mean(t * t, axis=-1, keepdims=True) - mean * mean
    y = lax.rsqrt(var + EPS)
    o_ref[...] = (t - mean) * y * g_ref[...] + b_ref[...]


def _tc_ln(t, pos_table, ln_gamma, ln_beta):
    pos3 = pos_table.reshape(1, S, D)
    g3 = ln_gamma.reshape(1, 1, D)
    b3 = ln_beta.reshape(1, 1, D)
    return pl.pallas_call(
        _ln_body,
        grid=(B // BB, S // SB),
        in_specs=[
            pl.BlockSpec((BB, SB, D), lambda i, j: (i, j, 0)),
            pl.BlockSpec((1, SB, D), lambda i, j: (0, j, 0)),
            pl.BlockSpec((1, 1, D), lambda i, j: (0, 0, 0)),
            pl.BlockSpec((1, 1, D), lambda i, j: (0, 0, 0)),
        ],
        out_specs=pl.BlockSpec((BB, SB, D), lambda i, j: (i, j, 0)),
        out_shape=jax.ShapeDtypeStruct((B, S, D), jnp.float32),
    )(t, pos3, g3, b3)


def kernel(x, id_table, pos_table, ln_gamma, ln_beta):
    rows = _sc_gather(x.reshape(N), id_table)
    return _tc_ln(rows.reshape(B, S, D), pos_table, ln_gamma, ln_beta)
